# baseline (speedup 1.0000x reference)
import jax
import jax.numpy as jnp
from jax import lax
from jax.experimental import pallas as pl
from jax.experimental.pallas import tpu as pltpu
from jax.experimental.pallas import tpu_sc as plsc

_N, _D, _R, _T = 100000, 128, 500, 200000
_REG = 0.01
_NC, _NS = 2, 16
_NW = _NC * _NS
_CB = 64
_NCHUNK = 100
_TPW = _CB * _NCHUNK
_TP = _NW * _TPW
_K0, _K1 = 192, 8
_PAD = _TP - _T
_DP = _D // 2
_NCH_TOT = _TP // _CB
_WROWS = 512
_IB = 5 * _CB

_GB = 10
_EB = _N // _GB
_SROWS = _TP // _D
_SB = _SROWS // _GB


def _score_body(epair, idx5_h, wpack_h, out,
                ibuf0, ibuf1, sbuf0, sbuf1, obuf0, obuf1, wbuf, tbuf,
                scbuf0, scbuf1,
                gsem0, gsem1, isem0, isem1, ssem0, ssem1):
    cax = lax.axis_index("c")
    sax = lax.axis_index("s")
    kk = jnp.where(cax == 0, _K0, _K1)
    khalf = jnp.where(cax == 0, _K0 // 2, _K1 // 2)
    cbase = jnp.where(cax == 0, sax * _K0, 16 * _K0 + sax * _K1)
    base = cbase * _CB

    ibufs = (ibuf0, ibuf1)
    sbufs = (sbuf0, sbuf1)
    obufs = (obuf0, obuf1)
    scbufs = (scbuf0, scbuf1)
    gsems = (gsem0, gsem1)
    isems = (isem0, isem1)
    ssems = (ssem0, ssem1)

    def idx_copy(c, b):
        return pltpu.make_async_copy(idx5_h.at[cbase + c], ibufs[b],
                                     isems[b])

    def gather_descs(b):
        ib = ibufs[b]
        return (
            pltpu.make_async_copy(epair.at[ib.at[pl.ds(0, _CB)]], sbufs[b],
                                  gsems[b]),
            pltpu.make_async_copy(epair.at[ib.at[pl.ds(2 * _CB, _CB)]],
                                  obufs[b], gsems[b]),
        )

    def fire(b):
        for dsc in gather_descs(b):
            dsc.start()

    def wait_gathers(b):
        for dsc in gather_descs(b):
            dsc.wait()

    lane = lax.iota(jnp.int32, 16)
    lane17 = lane * 17

    def compute(c, b):
        ib, sb, ob, scb = ibufs[b], sbufs[b], obufs[b], scbufs[b]
        def group(g, carry):
            relv = ib[pl.ds(_CB + g * 16, 16)] * _DP
            hv = ib[pl.ds(3 * _CB + g * 16, 16)]
            tv = ib[pl.ds(4 * _CB + g * 16, 16)]
            for rr in range(16):
                t = g * 16 + rr
                rbase, hbase, tbase = relv[rr], hv[rr], tv[rr]
                acc = jnp.zeros((16,), jnp.float32)
                for j in range(_D // 32):
                    s0 = plsc.bitcast(sb[t, pl.ds(hbase + j * 32, 16)],
                                      jnp.float32)
                    s1 = plsc.bitcast(sb[t, pl.ds(hbase + j * 32 + 16, 16)],
                                      jnp.float32)
                    o0 = plsc.bitcast(ob[t, pl.ds(tbase + j * 32, 16)],
                                      jnp.float32)
                    o1 = plsc.bitcast(ob[t, pl.ds(tbase + j * 32 + 16, 16)],
                                      jnp.float32)
                    rl, rh = plsc.unpack(
                        plsc.bitcast(wbuf[pl.ds(rbase + j * 16, 16)],
                                     jnp.bfloat16),
                        format=plsc.PackFormat.INTERLEAVED)
                    acc = acc + (s0 * o0) * rl + (s1 * o1) * rh
                tbuf[pl.ds(rr * 17, 16)] = acc

            s = jnp.zeros((16,), jnp.float32)
            for l in range(16):
                s = s + plsc.load_gather(tbuf, [lane17 + l])
            scb[pl.ds(g * 16, 16)] = s
            return carry

        lax.fori_loop(0, _CB // 16, group, 0)

    pltpu.sync_copy(wpack_h, wbuf)
    pltpu.sync_copy(idx5_h.at[cbase], ibuf0)
    fire(0)
    idx_copy(1, 1).start()

    def loop_body(i, carry):
        for b in (0, 1):
            c = 2 * i + b

            @pl.when(c + 1 < kk)
            def _():
                idx_copy(c + 1, 1 - b).wait()
                fire(1 - b)

            wait_gathers(b)

            @pl.when(c + 2 < kk)
            def _():
                idx_copy(c + 2, b).start()

            @pl.when(c >= 2)
            def _():
                pltpu.make_async_copy(scbufs[b], out.at[pl.ds(base, _CB)],
                                      ssems[b]).wait()

            compute(c, b)
            pltpu.make_async_copy(scbufs[b],
                                  out.at[pl.ds(base + c * _CB, _CB)],
                                  ssems[b]).start()
        return carry

    lax.fori_loop(0, khalf, loop_body, 0)

    pltpu.make_async_copy(scbuf0, out.at[pl.ds(base, _CB)], ssem0).wait()
    pltpu.make_async_copy(scbuf1, out.at[pl.ds(base, _CB)], ssem1).wait()


_score_call = pl.kernel(
    _score_body,
    out_type=jax.ShapeDtypeStruct((_TP,), jnp.float32),
    mesh=plsc.VectorSubcoreMesh(core_axis_name="c", subcore_axis_name="s",
                                num_cores=_NC, num_subcores=_NS),
    compiler_params=pltpu.CompilerParams(needs_layout_passes=False),
    scratch_types=[
        pltpu.VMEM((_IB,), jnp.int32),
        pltpu.VMEM((_IB,), jnp.int32),
        pltpu.VMEM((_CB, 2 * _D), jnp.int32),
        pltpu.VMEM((_CB, 2 * _D), jnp.int32),
        pltpu.VMEM((_CB, 2 * _D), jnp.int32),
        pltpu.VMEM((_CB, 2 * _D), jnp.int32),
        pltpu.VMEM((_WROWS * _DP,), jnp.int32),
        pltpu.VMEM((16 * 17,), jnp.float32),
        pltpu.VMEM((_CB,), jnp.float32),
        pltpu.VMEM((_CB,), jnp.float32),
        pltpu.SemaphoreType.DMA,
        pltpu.SemaphoreType.DMA,
        pltpu.SemaphoreType.DMA,
        pltpu.SemaphoreType.DMA,
        pltpu.SemaphoreType.DMA,
        pltpu.SemaphoreType.DMA,
    ],
)


def _loss_body(emb_ref, wrel_ref, sc_ref, y_ref, mk_ref, out_ref, acc_ref):
    i = pl.program_id(0)

    @pl.when(i == 0)
    def _():
        acc_ref[0] = 0.0
        acc_ref[1] = 0.0
        acc_ref[2] = jnp.sum(wrel_ref[...] ** 2)

    acc_ref[0] += jnp.sum(emb_ref[...] ** 2)
    s = sc_ref[...]
    y = y_ref[...]
    m = mk_ref[...]
    bce = jnp.maximum(s, 0.0) - s * y + jnp.log1p(jnp.exp(-jnp.abs(s)))
    acc_ref[1] += jnp.sum(m * bce)

    @pl.when(i == _GB - 1)
    def _():
        out_ref[0, 0] = (acc_ref[1] / _T
                         + _REG * (acc_ref[0] / (_N * _D)
                                   + acc_ref[2] / (_R * _D)))


_loss_call = pl.pallas_call(
    _loss_body,
    out_shape=jax.ShapeDtypeStruct((1, 1), jnp.float32),
    grid=(_GB,),
    in_specs=[
        pl.BlockSpec((_EB, _D), lambda i: (i, 0)),
        pl.BlockSpec((_WROWS, _D), lambda i: (0, 0)),
        pl.BlockSpec((_SB, _D), lambda i: (i, 0)),
        pl.BlockSpec((_SB, _D), lambda i: (i, 0)),
        pl.BlockSpec((_SB, _D), lambda i: (i, 0)),
    ],
    out_specs=pl.BlockSpec(memory_space=pltpu.SMEM),
    scratch_shapes=[pltpu.SMEM((4,), jnp.float32)],
)


def _pack_w(w):
    b = w.astype(jnp.bfloat16).reshape(_R, _D // 32, 2, 16)
    b = b.transpose(0, 1, 3, 2)
    pk = lax.bitcast_convert_type(b, jnp.int32).reshape(_R, _DP)
    return jnp.pad(pk, ((0, _WROWS - _R), (0, 0))).reshape(_WROWS * _DP)


def kernel(embed, heads, rels, tails, labels, w_relation):
    zpad = jnp.zeros((_PAD,), jnp.int32)
    hp = jnp.concatenate([heads, zpad])
    rp = jnp.concatenate([rels, zpad])
    tp = jnp.concatenate([tails, zpad])
    idx5 = jnp.stack([
        (hp >> 1).reshape(_NCH_TOT, _CB),
        rp.reshape(_NCH_TOT, _CB),
        (tp >> 1).reshape(_NCH_TOT, _CB),
        ((hp & 1) << 7).reshape(_NCH_TOT, _CB),
        ((tp & 1) << 7).reshape(_NCH_TOT, _CB),
    ], axis=1).reshape(_NCH_TOT, _IB)
    epair = lax.bitcast_convert_type(embed, jnp.int32).reshape(_N // 2, 2 * _D)
    wpack = _pack_w(w_relation)
    scores = _score_call(epair, idx5, wpack)

    y2 = jnp.pad(labels.astype(jnp.float32), (0, _PAD)).reshape(_SROWS, _D)
    m2 = (jnp.arange(_TP, dtype=jnp.int32) < _T).astype(
        jnp.float32).reshape(_SROWS, _D)
    s2 = scores.reshape(_SROWS, _D)
    w512 = jnp.pad(w_relation, ((0, _WROWS - _R), (0, 0)))
    out = _loss_call(embed, w512, s2, y2, m2)
    return out[0, 0]


# speedup vs baseline: 1.0174x; 1.0174x over previous
import jax
import jax.numpy as jnp
from jax import lax
from jax.experimental import pallas as pl
from jax.experimental.pallas import tpu as pltpu
from jax.experimental.pallas import tpu_sc as plsc

_N, _D, _R, _T = 100000, 128, 500, 200000
_REG = 0.01
_NC, _NS = 2, 16
_NW = _NC * _NS
_CB = 64
_NCHUNK = 100
_TPW = _CB * _NCHUNK
_TP = _NW * _TPW
_K0, _K1 = 180, 20
_PAD = _TP - _T
_DP = _D // 2
_NCH_TOT = _TP // _CB
_WROWS = 512
_IB = 5 * _CB

_GB = 10
_EB = _N // _GB
_SROWS = _TP // _D
_SB = _SROWS // _GB


def _score_body(epair, idx5_h, wpack_h, out,
                ibuf0, ibuf1, sbuf0, sbuf1, obuf0, obuf1, wbuf, tbuf,
                scbuf0, scbuf1,
                gsem0, gsem1, isem0, isem1, ssem0, ssem1):
    cax = lax.axis_index("c")
    sax = lax.axis_index("s")
    kk = jnp.where(cax == 0, _K0, _K1)
    khalf = jnp.where(cax == 0, _K0 // 2, _K1 // 2)
    cbase = jnp.where(cax == 0, sax * _K0, 16 * _K0 + sax * _K1)
    base = cbase * _CB

    ibufs = (ibuf0, ibuf1)
    sbufs = (sbuf0, sbuf1)
    obufs = (obuf0, obuf1)
    scbufs = (scbuf0, scbuf1)
    gsems = (gsem0, gsem1)
    isems = (isem0, isem1)
    ssems = (ssem0, ssem1)

    def idx_copy(c, b):
        return pltpu.make_async_copy(idx5_h.at[cbase + c], ibufs[b],
                                     isems[b])

    def gather_descs(b):
        ib = ibufs[b]
        return (
            pltpu.make_async_copy(epair.at[ib.at[pl.ds(0, _CB)]], sbufs[b],
                                  gsems[b]),
            pltpu.make_async_copy(epair.at[ib.at[pl.ds(2 * _CB, _CB)]],
                                  obufs[b], gsems[b]),
        )

    def fire(b):
        for dsc in gather_descs(b):
            dsc.start()

    def wait_gathers(b):
        for dsc in gather_descs(b):
            dsc.wait()

    lane = lax.iota(jnp.int32, 16)
    lane17 = lane * 17

    def compute(c, b):
        ib, sb, ob, scb = ibufs[b], sbufs[b], obufs[b], scbufs[b]
        def group(g, carry):
            relv = ib[pl.ds(_CB + g * 16, 16)] * _DP
            hv = ib[pl.ds(3 * _CB + g * 16, 16)]
            tv = ib[pl.ds(4 * _CB + g * 16, 16)]
            for rr in range(16):
                t = g * 16 + rr
                rbase, hbase, tbase = relv[rr], hv[rr], tv[rr]
                acc = jnp.zeros((16,), jnp.float32)
                for j in range(_D // 32):
                    s0 = plsc.bitcast(sb[t, pl.ds(hbase + j * 32, 16)],
                                      jnp.float32)
                    s1 = plsc.bitcast(sb[t, pl.ds(hbase + j * 32 + 16, 16)],
                                      jnp.float32)
                    o0 = plsc.bitcast(ob[t, pl.ds(tbase + j * 32, 16)],
                                      jnp.float32)
                    o1 = plsc.bitcast(ob[t, pl.ds(tbase + j * 32 + 16, 16)],
                                      jnp.float32)
                    rl, rh = plsc.unpack(
                        plsc.bitcast(wbuf[pl.ds(rbase + j * 16, 16)],
                                     jnp.bfloat16),
                        format=plsc.PackFormat.INTERLEAVED)
                    acc = acc + (s0 * o0) * rl + (s1 * o1) * rh
                tbuf[pl.ds(rr * 17, 16)] = acc

            s = jnp.zeros((16,), jnp.float32)
            for l in range(16):
                s = s + plsc.load_gather(tbuf, [lane17 + l])
            scb[pl.ds(g * 16, 16)] = s
            return carry

        lax.fori_loop(0, _CB // 16, group, 0)

    pltpu.sync_copy(wpack_h, wbuf)
    pltpu.sync_copy(idx5_h.at[cbase], ibuf0)
    fire(0)
    idx_copy(1, 1).start()

    def loop_body(i, carry):
        for b in (0, 1):
            c = 2 * i + b

            @pl.when(c + 1 < kk)
            def _():
                idx_copy(c + 1, 1 - b).wait()
                fire(1 - b)

            wait_gathers(b)

            @pl.when(c + 2 < kk)
            def _():
                idx_copy(c + 2, b).start()

            @pl.when(c >= 2)
            def _():
                pltpu.make_async_copy(scbufs[b], out.at[pl.ds(base, _CB)],
                                      ssems[b]).wait()

            compute(c, b)
            pltpu.make_async_copy(scbufs[b],
                                  out.at[pl.ds(base + c * _CB, _CB)],
                                  ssems[b]).start()
        return carry

    lax.fori_loop(0, khalf, loop_body, 0)

    pltpu.make_async_copy(scbuf0, out.at[pl.ds(base, _CB)], ssem0).wait()
    pltpu.make_async_copy(scbuf1, out.at[pl.ds(base, _CB)], ssem1).wait()


_score_call = pl.kernel(
    _score_body,
    out_type=jax.ShapeDtypeStruct((_TP,), jnp.float32),
    mesh=plsc.VectorSubcoreMesh(core_axis_name="c", subcore_axis_name="s",
                                num_cores=_NC, num_subcores=_NS),
    compiler_params=pltpu.CompilerParams(needs_layout_passes=False),
    scratch_types=[
        pltpu.VMEM((_IB,), jnp.int32),
        pltpu.VMEM((_IB,), jnp.int32),
        pltpu.VMEM((_CB, 2 * _D), jnp.int32),
        pltpu.VMEM((_CB, 2 * _D), jnp.int32),
        pltpu.VMEM((_CB, 2 * _D), jnp.int32),
        pltpu.VMEM((_CB, 2 * _D), jnp.int32),
        pltpu.VMEM((_WROWS * _DP,), jnp.int32),
        pltpu.VMEM((16 * 17,), jnp.float32),
        pltpu.VMEM((_CB,), jnp.float32),
        pltpu.VMEM((_CB,), jnp.float32),
        pltpu.SemaphoreType.DMA,
        pltpu.SemaphoreType.DMA,
        pltpu.SemaphoreType.DMA,
        pltpu.SemaphoreType.DMA,
        pltpu.SemaphoreType.DMA,
        pltpu.SemaphoreType.DMA,
    ],
)


def _reg_body(emb_ref, wrel_ref, out_ref, acc_ref):
    i = pl.program_id(0)

    @pl.when(i == 0)
    def _():
        acc_ref[0] = jnp.sum(wrel_ref[...] ** 2) / (_R * _D)

    acc_ref[0] += jnp.sum(emb_ref[...] ** 2) / (_N * _D)

    @pl.when(i == _GB - 1)
    def _():
        out_ref[0, 0] = acc_ref[0]


_reg_call = pl.pallas_call(
    _reg_body,
    out_shape=jax.ShapeDtypeStruct((1, 1), jnp.float32),
    grid=(_GB,),
    in_specs=[
        pl.BlockSpec((_EB, _D), lambda i: (i, 0)),
        pl.BlockSpec((_WROWS, _D), lambda i: (0, 0)),
    ],
    out_specs=pl.BlockSpec(memory_space=pltpu.SMEM),
    scratch_shapes=[pltpu.SMEM((1,), jnp.float32)],
)


def _bce_body(reg_ref, sc_ref, y_ref, mk_ref, out_ref):
    s = sc_ref[...]
    y = y_ref[...]
    m = mk_ref[...]
    bce = jnp.maximum(s, 0.0) - s * y + jnp.log1p(jnp.exp(-jnp.abs(s)))
    out_ref[0, 0] = jnp.sum(m * bce) / _T + _REG * reg_ref[0, 0]


_bce_call = pl.pallas_call(
    _bce_body,
    out_shape=jax.ShapeDtypeStruct((1, 1), jnp.float32),
    in_specs=[
        pl.BlockSpec(memory_space=pltpu.SMEM),
        pl.BlockSpec((_SROWS, _D), lambda: (0, 0)),
        pl.BlockSpec((_SROWS, _D), lambda: (0, 0)),
        pl.BlockSpec((_SROWS, _D), lambda: (0, 0)),
    ],
    out_specs=pl.BlockSpec(memory_space=pltpu.SMEM),
)


def _pack_w(w):
    b = w.astype(jnp.bfloat16).reshape(_R, _D // 32, 2, 16)
    b = b.transpose(0, 1, 3, 2)
    pk = lax.bitcast_convert_type(b, jnp.int32).reshape(_R, _DP)
    return jnp.pad(pk, ((0, _WROWS - _R), (0, 0))).reshape(_WROWS * _DP)


def kernel(embed, heads, rels, tails, labels, w_relation):
    zpad = jnp.zeros((_PAD,), jnp.int32)
    hp = jnp.concatenate([heads, zpad])
    rp = jnp.concatenate([rels, zpad])
    tp = jnp.concatenate([tails, zpad])
    idx5 = jnp.stack([
        (hp >> 1).reshape(_NCH_TOT, _CB),
        rp.reshape(_NCH_TOT, _CB),
        (tp >> 1).reshape(_NCH_TOT, _CB),
        ((hp & 1) << 7).reshape(_NCH_TOT, _CB),
        ((tp & 1) << 7).reshape(_NCH_TOT, _CB),
    ], axis=1).reshape(_NCH_TOT, _IB)
    epair = lax.bitcast_convert_type(embed, jnp.int32).reshape(_N // 2, 2 * _D)
    wpack = _pack_w(w_relation)
    scores = _score_call(epair, idx5, wpack)

    y2 = jnp.pad(labels.astype(jnp.float32), (0, _PAD)).reshape(_SROWS, _D)
    m2 = (jnp.arange(_TP, dtype=jnp.int32) < _T).astype(
        jnp.float32).reshape(_SROWS, _D)
    s2 = scores.reshape(_SROWS, _D)
    w512 = jnp.pad(w_relation, ((0, _WROWS - _R), (0, 0)))
    reg = _reg_call(embed, w512)
    out = _bce_call(reg, s2, y2, m2)
    return out[0, 0]
